# baseline (speedup 1.0000x reference)
import functools

import jax
import jax.numpy as jnp
from jax import lax
from jax.experimental import pallas as pl
from jax.experimental.pallas import tpu as pltpu
from jax.experimental.pallas import tpu_sc as plsc


def _make_gather(B: int, S: int, V: int, D: int):
    info = plsc.get_sparse_core_info()
    NC, NS = info.num_cores, info.num_subcores
    NW = NC * NS
    N = B * S
    rows_per_w = N // NW
    C = 8
    NBUF = 15
    n_chunks = rows_per_w // C
    M = NBUF * ((n_chunks - NBUF) // NBUF)
    T = n_chunks - M
    assert NBUF <= T < 2 * NBUF and M > 0 and M % NBUF == 0
    assert S % rows_per_w == 0

    w_per_b = S // rows_per_w

    mesh = plsc.VectorSubcoreMesh(core_axis_name="c", subcore_axis_name="s")

    @functools.partial(
        pl.kernel,
        mesh=mesh,
        out_type=jax.ShapeDtypeStruct((B, S, D), jnp.float32),
        scratch_types=[
            pltpu.VMEM((rows_per_w,), jnp.int32),
            pltpu.VMEM((NBUF, C, D), jnp.float32),
            pltpu.SemaphoreType.DMA((NBUF,)),
            pltpu.SemaphoreType.DMA((NBUF,)),
        ],
    )
    def gather_kernel(table_hbm, idx_hbm, out_hbm, idx_v, rows_v, gsem, wsem):
        wid = lax.axis_index("s") * NC + lax.axis_index("c")
        bi = wid // w_per_b
        base = (wid % w_per_b) * rows_per_w

        def gather_start(j, b):
            pltpu.async_copy(
                table_hbm.at[idx_v.at[pl.ds(j * C, C)]], rows_v.at[b],
                gsem.at[b])

        def gather_wait(j, b):
            pltpu.make_async_copy(
                table_hbm.at[idx_v.at[pl.ds(j * C, C)]], rows_v.at[b],
                gsem.at[b]).wait()

        def write_start(j, b):
            pltpu.async_copy(
                rows_v.at[b], out_hbm.at[bi, pl.ds(base + j * C, C)],
                wsem.at[b])

        def write_wait(j, b):
            pltpu.make_async_copy(
                rows_v.at[b], out_hbm.at[bi, pl.ds(base + j * C, C)],
                wsem.at[b]).wait()

        pltpu.sync_copy(idx_hbm.at[bi, pl.ds(base, rows_per_w)], idx_v)
        for b in range(NBUF):
            gather_start(b, b)
        def round_body(g):
            for b in range(NBUF):
                gather_wait(g + b, b)
                write_start(g + b, b)
            for b in range(NBUF):
                write_wait(g + b, b)
                gather_start(g + b + NBUF, b)
        pl.loop(0, M, step=NBUF)(round_body)
        for t in range(NBUF):
            gather_wait(M + t, t)
            write_start(M + t, t)
        for t in range(NBUF, T):
            s = t - NBUF
            write_wait(M + s, s)
            gather_start(M + t, s)
        for t in range(NBUF, T):
            s = t - NBUF
            gather_wait(M + t, s)
            write_start(M + t, s)
        for s in range(T - NBUF):
            write_wait(M + NBUF + s, s)
        for s in range(T - NBUF, NBUF):
            write_wait(M + s, s)

    return gather_kernel


def kernel(input_ids, word_embeddings):
    B, S = input_ids.shape
    V, D = word_embeddings.shape
    ids = input_ids.astype(jnp.int32)
    return _make_gather(B, S, V, D)(word_embeddings, ids)


# speedup vs baseline: 1.0226x; 1.0226x over previous
import functools

import jax
import jax.numpy as jnp
from jax import lax
from jax.experimental import pallas as pl
from jax.experimental.pallas import tpu as pltpu
from jax.experimental.pallas import tpu_sc as plsc


def _make_gather(B: int, S: int, V: int, D: int):
    info = plsc.get_sparse_core_info()
    NC, NS = info.num_cores, info.num_subcores
    NW = NC * NS
    N = B * S
    rows_per_w = N // NW
    C = 8
    NBUF = 10
    n_chunks = rows_per_w // C
    M = NBUF * ((n_chunks - NBUF) // NBUF)
    T = n_chunks - M
    assert NBUF <= T < 2 * NBUF and M > 0 and M % NBUF == 0
    assert S % rows_per_w == 0

    w_per_b = S // rows_per_w

    mesh = plsc.VectorSubcoreMesh(core_axis_name="c", subcore_axis_name="s")

    @functools.partial(
        pl.kernel,
        mesh=mesh,
        out_type=jax.ShapeDtypeStruct((B, S, D), jnp.float32),
        scratch_types=[
            pltpu.VMEM((rows_per_w,), jnp.int32),
            pltpu.VMEM((NBUF, C, D), jnp.float32),
            pltpu.SemaphoreType.DMA((NBUF,)),
            pltpu.SemaphoreType.DMA((NBUF,)),
        ],
    )
    def gather_kernel(table_hbm, idx_hbm, out_hbm, idx_v, rows_v, gsem, wsem):
        wid = lax.axis_index("s") * NC + lax.axis_index("c")
        bi = wid // w_per_b
        base = (wid % w_per_b) * rows_per_w

        def gather_start(j, b):
            pltpu.async_copy(
                table_hbm.at[idx_v.at[pl.ds(j * C, C)]], rows_v.at[b],
                gsem.at[b])

        def gather_wait(j, b):
            pltpu.make_async_copy(
                table_hbm.at[idx_v.at[pl.ds(j * C, C)]], rows_v.at[b],
                gsem.at[b]).wait()

        def write_start(j, b):
            pltpu.async_copy(
                rows_v.at[b], out_hbm.at[bi, pl.ds(base + j * C, C)],
                wsem.at[b])

        def write_wait(j, b):
            pltpu.make_async_copy(
                rows_v.at[b], out_hbm.at[bi, pl.ds(base + j * C, C)],
                wsem.at[b]).wait()

        pltpu.sync_copy(idx_hbm.at[bi, pl.ds(base, rows_per_w)], idx_v)
        for b in range(NBUF):
            gather_start(b, b)
        def round_body(g):
            for b in range(NBUF):
                gather_wait(g + b, b)
                write_start(g + b, b)
            for b in range(NBUF):
                write_wait(g + b, b)
                gather_start(g + b + NBUF, b)
        pl.loop(0, M, step=NBUF)(round_body)
        for t in range(NBUF):
            gather_wait(M + t, t)
            write_start(M + t, t)
        for t in range(NBUF, T):
            s = t - NBUF
            write_wait(M + s, s)
            gather_start(M + t, s)
        for t in range(NBUF, T):
            s = t - NBUF
            gather_wait(M + t, s)
            write_start(M + t, s)
        for s in range(T - NBUF):
            write_wait(M + NBUF + s, s)
        for s in range(T - NBUF, NBUF):
            write_wait(M + s, s)

    return gather_kernel


def kernel(input_ids, word_embeddings):
    B, S = input_ids.shape
    V, D = word_embeddings.shape
    ids = input_ids.astype(jnp.int32)
    return _make_gather(B, S, V, D)(word_embeddings, ids)


# speedup vs baseline: 1.0324x; 1.0096x over previous
import functools

import jax
import jax.numpy as jnp
from jax import lax
from jax.experimental import pallas as pl
from jax.experimental.pallas import tpu as pltpu
from jax.experimental.pallas import tpu_sc as plsc


def _make_gather(B: int, S: int, V: int, D: int):
    info = plsc.get_sparse_core_info()
    NC, NS = info.num_cores, info.num_subcores
    NW = NC * NS
    N = B * S
    rows_per_w = N // NW
    C = 8
    NBUF = 8
    n_chunks = rows_per_w // C
    M = NBUF * ((n_chunks - NBUF) // NBUF)
    T = n_chunks - M
    assert NBUF <= T < 2 * NBUF and M > 0 and M % NBUF == 0
    assert S % rows_per_w == 0

    w_per_b = S // rows_per_w

    mesh = plsc.VectorSubcoreMesh(core_axis_name="c", subcore_axis_name="s")

    @functools.partial(
        pl.kernel,
        mesh=mesh,
        out_type=jax.ShapeDtypeStruct((B, S, D), jnp.float32),
        scratch_types=[
            pltpu.VMEM((rows_per_w,), jnp.int32),
            pltpu.VMEM((NBUF * C, D), jnp.float32),
            pltpu.SemaphoreType.DMA((NBUF,)),
            pltpu.SemaphoreType.DMA((NBUF,)),
        ],
    )
    def gather_kernel(table_hbm, idx_hbm, out_hbm, idx_v, rows_v, gsem, wsem):
        wid = lax.axis_index("s") * NC + lax.axis_index("c")
        bi = wid // w_per_b
        base = (wid % w_per_b) * rows_per_w

        def gather_start(j, b):
            pltpu.async_copy(
                table_hbm.at[idx_v.at[pl.ds(j * C, C)]],
                rows_v.at[pl.ds(b * C, C)], gsem.at[b])

        def gather_wait(j, b):
            pltpu.make_async_copy(
                table_hbm.at[idx_v.at[pl.ds(j * C, C)]],
                rows_v.at[pl.ds(b * C, C)], gsem.at[b]).wait()

        def write_start(j, b, k=1):
            pltpu.async_copy(
                rows_v.at[pl.ds(b * C, k * C)],
                out_hbm.at[bi, pl.ds(base + j * C, k * C)], wsem.at[b])

        def write_wait(j, b, k=1):
            pltpu.make_async_copy(
                rows_v.at[pl.ds(b * C, k * C)],
                out_hbm.at[bi, pl.ds(base + j * C, k * C)], wsem.at[b]).wait()

        pltpu.sync_copy(idx_hbm.at[bi, pl.ds(base, rows_per_w)], idx_v)
        for b in range(NBUF):
            gather_start(b, b)
        def round_body(g):
            for p in range(0, NBUF, 2):
                gather_wait(g + p, p)
                gather_wait(g + p + 1, p + 1)
                write_start(g + p, p, k=2)
            for p in range(0, NBUF, 2):
                write_wait(g + p, p, k=2)
                gather_start(g + p + NBUF, p)
                gather_start(g + p + 1 + NBUF, p + 1)
        pl.loop(0, M, step=NBUF)(round_body)
        for t in range(NBUF):
            gather_wait(M + t, t)
            write_start(M + t, t)
        for t in range(NBUF, T):
            s = t - NBUF
            write_wait(M + s, s)
            gather_start(M + t, s)
        for t in range(NBUF, T):
            s = t - NBUF
            gather_wait(M + t, s)
            write_start(M + t, s)
        for s in range(T - NBUF):
            write_wait(M + NBUF + s, s)
        for s in range(T - NBUF, NBUF):
            write_wait(M + s, s)

    return gather_kernel


def kernel(input_ids, word_embeddings):
    B, S = input_ids.shape
    V, D = word_embeddings.shape
    ids = input_ids.astype(jnp.int32)
    return _make_gather(B, S, V, D)(word_embeddings, ids)


# speedup vs baseline: 1.0338x; 1.0013x over previous
import functools

import jax
import jax.numpy as jnp
from jax import lax
from jax.experimental import pallas as pl
from jax.experimental.pallas import tpu as pltpu
from jax.experimental.pallas import tpu_sc as plsc


def _make_gather(B: int, S: int, V: int, D: int):
    info = plsc.get_sparse_core_info()
    NC, NS = info.num_cores, info.num_subcores
    NW = NC * NS
    N = B * S
    rows_per_w = N // NW
    C = 8
    NBUF = 8
    n_chunks = rows_per_w // C
    M = NBUF * ((n_chunks - NBUF) // NBUF)
    T = n_chunks - M
    assert NBUF <= T < 2 * NBUF and M > 0 and M % NBUF == 0
    assert S % rows_per_w == 0

    w_per_b = S // rows_per_w

    mesh = plsc.VectorSubcoreMesh(core_axis_name="c", subcore_axis_name="s")

    @functools.partial(
        pl.kernel,
        mesh=mesh,
        out_type=jax.ShapeDtypeStruct((B, S, D), jnp.float32),
        scratch_types=[
            pltpu.VMEM((rows_per_w,), jnp.int32),
            pltpu.VMEM((NBUF * C, D), jnp.float32),
            pltpu.SemaphoreType.DMA((NBUF,)),
            pltpu.SemaphoreType.DMA((NBUF,)),
        ],
    )
    def gather_kernel(table_hbm, idx_hbm, out_hbm, idx_v, rows_v, gsem, wsem):
        wid = lax.axis_index("s") * NC + lax.axis_index("c")
        bi = wid // w_per_b
        base = (wid % w_per_b) * rows_per_w

        def gather_start(j, b):
            pltpu.async_copy(
                table_hbm.at[idx_v.at[pl.ds(j * C, C)]],
                rows_v.at[pl.ds(b * C, C)], gsem.at[b])

        def gather_wait(j, b):
            pltpu.make_async_copy(
                table_hbm.at[idx_v.at[pl.ds(j * C, C)]],
                rows_v.at[pl.ds(b * C, C)], gsem.at[b]).wait()

        def write_start(j, b, k=1):
            pltpu.async_copy(
                rows_v.at[pl.ds(b * C, k * C)],
                out_hbm.at[bi, pl.ds(base + j * C, k * C)], wsem.at[b])

        def write_wait(j, b, k=1):
            pltpu.make_async_copy(
                rows_v.at[pl.ds(b * C, k * C)],
                out_hbm.at[bi, pl.ds(base + j * C, k * C)], wsem.at[b]).wait()

        pltpu.sync_copy(idx_hbm.at[bi, pl.ds(base, rows_per_w)], idx_v)
        for b in range(NBUF):
            gather_start(b, b)
        def round_body(g):
            for p in range(0, NBUF, 4):
                for q in range(4):
                    gather_wait(g + p + q, p + q)
                write_start(g + p, p, k=4)
            for p in range(0, NBUF, 4):
                write_wait(g + p, p, k=4)
                for q in range(4):
                    gather_start(g + p + q + NBUF, p + q)
        pl.loop(0, M, step=NBUF)(round_body)
        for t in range(NBUF):
            gather_wait(M + t, t)
            write_start(M + t, t)
        for t in range(NBUF, T):
            s = t - NBUF
            write_wait(M + s, s)
            gather_start(M + t, s)
        for t in range(NBUF, T):
            s = t - NBUF
            gather_wait(M + t, s)
            write_start(M + t, s)
        for s in range(T - NBUF):
            write_wait(M + NBUF + s, s)
        for s in range(T - NBUF, NBUF):
            write_wait(M + s, s)

    return gather_kernel


def kernel(input_ids, word_embeddings):
    B, S = input_ids.shape
    V, D = word_embeddings.shape
    ids = input_ids.astype(jnp.int32)
    return _make_gather(B, S, V, D)(word_embeddings, ids)


# speedup vs baseline: 1.0341x; 1.0003x over previous
import functools

import jax
import jax.numpy as jnp
from jax import lax
from jax.experimental import pallas as pl
from jax.experimental.pallas import tpu as pltpu
from jax.experimental.pallas import tpu_sc as plsc


def _make_gather(B: int, S: int, V: int, D: int):
    info = plsc.get_sparse_core_info()
    NC, NS = info.num_cores, info.num_subcores
    NW = NC * NS
    N = B * S
    rows_per_w = N // NW
    C = 8
    NBUF = 8
    n_chunks = rows_per_w // C
    M = NBUF * ((n_chunks - NBUF) // NBUF)
    T = n_chunks - M
    assert NBUF <= T < 2 * NBUF and M > 0 and M % NBUF == 0
    assert S % rows_per_w == 0

    w_per_b = S // rows_per_w

    mesh = plsc.VectorSubcoreMesh(core_axis_name="c", subcore_axis_name="s")

    @functools.partial(
        pl.kernel,
        mesh=mesh,
        out_type=jax.ShapeDtypeStruct((B, S, D), jnp.float32),
        scratch_types=[
            pltpu.VMEM((rows_per_w,), jnp.int32),
            pltpu.VMEM((NBUF * C, D), jnp.float32),
            pltpu.SemaphoreType.DMA((NBUF,)),
            pltpu.SemaphoreType.DMA((NBUF,)),
        ],
    )
    def gather_kernel(table_hbm, idx_hbm, out_hbm, idx_v, rows_v, gsem, wsem):
        wid = lax.axis_index("s") * NC + lax.axis_index("c")
        bi = wid // w_per_b
        base = (wid % w_per_b) * rows_per_w

        def gather_start(j, b):
            pltpu.async_copy(
                table_hbm.at[idx_v.at[pl.ds(j * C, C)]],
                rows_v.at[pl.ds(b * C, C)], gsem.at[b])

        def gather_wait(j, b):
            pltpu.make_async_copy(
                table_hbm.at[idx_v.at[pl.ds(j * C, C)]],
                rows_v.at[pl.ds(b * C, C)], gsem.at[b]).wait()

        def write_start(j, b, k=1):
            pltpu.async_copy(
                rows_v.at[pl.ds(b * C, k * C)],
                out_hbm.at[bi, pl.ds(base + j * C, k * C)], wsem.at[b])

        def write_wait(j, b, k=1):
            pltpu.make_async_copy(
                rows_v.at[pl.ds(b * C, k * C)],
                out_hbm.at[bi, pl.ds(base + j * C, k * C)], wsem.at[b]).wait()

        pltpu.sync_copy(idx_hbm.at[bi, pl.ds(base, rows_per_w)], idx_v)
        for b in range(NBUF):
            gather_start(b, b)
        def round_body(g):
            for p in range(0, NBUF, 2):
                gather_wait(g + p, p)
                gather_wait(g + p + 1, p + 1)
                write_start(g + p, p, k=2)
            for p in range(0, NBUF, 2):
                write_wait(g + p, p, k=2)
                gather_start(g + p + NBUF, p)
                gather_start(g + p + 1 + NBUF, p + 1)
        pl.loop(0, M, step=NBUF)(round_body)
        for t in range(NBUF):
            gather_wait(M + t, t)
            write_start(M + t, t)
        for t in range(NBUF, T):
            s = t - NBUF
            write_wait(M + s, s)
            gather_start(M + t, s)
        for t in range(NBUF, T):
            s = t - NBUF
            gather_wait(M + t, s)
            write_start(M + t, s)
        for s in range(T - NBUF):
            write_wait(M + NBUF + s, s)
        for s in range(T - NBUF, NBUF):
            write_wait(M + s, s)

    return gather_kernel


def kernel(input_ids, word_embeddings):
    B, S = input_ids.shape
    V, D = word_embeddings.shape
    ids = input_ids.astype(jnp.int32)
    return _make_gather(B, S, V, D)(word_embeddings, ids)
